# baseline (speedup 1.0000x reference)
import dataclasses
import functools

import jax
import jax.numpy as jnp
from jax import lax
from jax.experimental import pallas as pl
from jax.experimental.pallas import tpu as pltpu
from jax.experimental.pallas import tpu_sc as plsc

_L = 200
_D = 32
_B = 4096
_NW = 32
_NB = 128
_NITEMS = _L * (_B // _NB)
_IPW = _NITEMS // _NW
_LG = _L // 8


def _compiler_params():
    cp = pltpu.CompilerParams(use_tc_tiling_on_sc=False)
    if "needs_layout_passes" in pltpu.CompilerParams.__dataclass_fields__:
        cp = dataclasses.replace(cp, needs_layout_passes=False)
    return cp


def _sc_embed(xv, token_table, pos_table):
    mesh = plsc.VectorSubcoreMesh(core_axis_name="c", subcore_axis_name="s")

    @functools.partial(
        pl.kernel,
        out_type=jax.ShapeDtypeStruct((_L, _D // 8, _B // _NB, 8, _NB),
                                      jnp.float32),
        mesh=mesh,
        compiler_params=_compiler_params(),
        scratch_types=[
            pltpu.VMEM((_L, _D), jnp.float32),
            pltpu.VMEM((2, _NB), jnp.int32),
            pltpu.VMEM((2, _NB, _D), jnp.float32),
            pltpu.VMEM((2, _D, _NB), jnp.float32),
            pltpu.SemaphoreType.DMA,
            pltpu.SemaphoreType.DMA,
            pltpu.SemaphoreType.DMA,
        ],
    )
    def k(x_hbm, tok_hbm, pos_hbm, out_hbm, pos_v, idx_v, g_v, out_v,
          isem, gsem, osem):
        wid = lax.axis_index("s") * 2 + lax.axis_index("c")
        q0 = wid * _IPW
        pltpu.sync_copy(pos_hbm, pos_v)

        row_lo = lax.iota(jnp.int32, 16)
        row_hi = row_lo + 16

        def item_coords(i):
            q = q0 + i
            l = q // (_B // _NB)
            bg = q % (_B // _NB)
            return l, bg

        def fire_idx(i, b):
            l, bg = item_coords(i)
            pltpu.async_copy(x_hbm.at[l // 8, bg, l % 8], idx_v.at[b], isem)

        def fire_gather(b):
            pltpu.async_copy(tok_hbm.at[idx_v.at[b]], g_v.at[b], gsem)

        def drain(dummy_src, dst, sem):
            pltpu.make_async_copy(dummy_src, dst, sem).wait()

        for b in (0, 1):
            fire_idx(b, b)
        for b in (0, 1):
            drain(x_hbm.at[0, 0, 0], idx_v.at[b], isem)
            fire_gather(b)

        @pl.loop(0, _IPW, step=2)
        def _slot(i0):
            for b in (0, 1):
                i = i0 + b
                l, bg = item_coords(i)
                gb = g_v.at[b]
                ob = out_v.at[b]

                drain(tok_hbm.at[idx_v.at[b]], gb, gsem)
                @pl.when(i >= 2)
                def _():
                    for _g in range(4):
                        drain(tok_hbm.at[idx_v.at[b]],
                              ob.at[pl.ds(0, 8)], osem)

                pv0 = pos_v[l, pl.ds(0, 16)]
                pv1 = pos_v[l, pl.ds(16, 16)]

                @pl.loop(0, _NB, step=4)
                def _rows(r0):
                    for u in range(4):
                        r = r0 + u
                        col = jnp.full((16,), r, jnp.int32)
                        v0 = gb[r, pl.ds(0, 16)] + pv0
                        v1 = gb[r, pl.ds(16, 16)] + pv1
                        plsc.store_scatter(ob, [row_lo, col], v0)
                        plsc.store_scatter(ob, [row_hi, col], v1)

                for g in range(4):
                    pltpu.async_copy(ob.at[pl.ds(8 * g, 8)],
                                     out_hbm.at[l, g, bg], osem)

                @pl.when(i + 2 < _IPW)
                def _():
                    fire_idx(i + 2, b)
                    drain(x_hbm.at[0, 0, 0], idx_v.at[b], isem)
                    fire_gather(b)

        for b in (0, 1):
            for _g in range(4):
                drain(tok_hbm.at[idx_v.at[b]],
                      out_v.at[b].at[pl.ds(0, 8)], osem)

    return k(xv, token_table, pos_table)


def kernel(x, token_table, pos_table):
    b, l = x.shape
    xi = x.astype(jnp.int32)
    xv = jnp.transpose(xi.reshape(_B // _NB, _NB, _LG, 8), (2, 0, 3, 1))
    out5 = _sc_embed(xv, token_table, pos_table)
    return jnp.transpose(out5, (2, 4, 0, 1, 3)).reshape(b, l, _D)


# speedup vs baseline: 1.4706x; 1.4706x over previous
import dataclasses
import functools

import jax
import jax.numpy as jnp
from jax import lax
from jax.experimental import pallas as pl
from jax.experimental.pallas import tpu as pltpu
from jax.experimental.pallas import tpu_sc as plsc

_L = 200
_D = 32
_B = 4096
_NW = 32
_NB = 128
_NITEMS = _L * (_B // _NB)
_IPW = _NITEMS // _NW
_LG = _L // 8


def _compiler_params():
    cp = pltpu.CompilerParams(use_tc_tiling_on_sc=False)
    if "needs_layout_passes" in pltpu.CompilerParams.__dataclass_fields__:
        cp = dataclasses.replace(cp, needs_layout_passes=False)
    return cp


def _sc_embed(xv, token_table, pos_table):
    mesh = plsc.VectorSubcoreMesh(core_axis_name="c", subcore_axis_name="s")

    @functools.partial(
        pl.kernel,
        out_type=jax.ShapeDtypeStruct((_L, _D // 8, _B // _NB, 8, _NB),
                                      jnp.float32),
        mesh=mesh,
        compiler_params=_compiler_params(),
        scratch_types=[
            pltpu.VMEM((_L, _D), jnp.float32),
            pltpu.VMEM((2, _NB), jnp.int32),
            pltpu.VMEM((2, _NB, _D), jnp.float32),
            pltpu.VMEM((2, _D, _NB + 1), jnp.float32),
            pltpu.SemaphoreType.DMA,
            pltpu.SemaphoreType.DMA,
            pltpu.SemaphoreType.DMA,
        ],
    )
    def k(x_hbm, tok_hbm, pos_hbm, out_hbm, pos_v, idx_v, g_v, out_v,
          isem, gsem, osem):
        wid = lax.axis_index("s") * 2 + lax.axis_index("c")
        q0 = wid * _IPW
        pltpu.sync_copy(pos_hbm, pos_v)

        row_lo = lax.iota(jnp.int32, 16)
        row_hi = row_lo + 16

        def item_coords(i):
            q = q0 + i
            l = q // (_B // _NB)
            bg = q % (_B // _NB)
            return l, bg

        def fire_idx(i, b):
            l, bg = item_coords(i)
            pltpu.async_copy(x_hbm.at[l // 8, bg, l % 8], idx_v.at[b], isem)

        def fire_gather(b):
            pltpu.async_copy(tok_hbm.at[idx_v.at[b]], g_v.at[b], gsem)

        def drain(dummy_src, dst, sem):
            pltpu.make_async_copy(dummy_src, dst, sem).wait()

        for b in (0, 1):
            fire_idx(b, b)
        for b in (0, 1):
            drain(x_hbm.at[0, 0, 0], idx_v.at[b], isem)
            fire_gather(b)

        @pl.loop(0, _IPW, step=2)
        def _slot(i0):
            for b in (0, 1):
                i = i0 + b
                l, bg = item_coords(i)
                gb = g_v.at[b]
                ob = out_v.at[b]

                drain(tok_hbm.at[idx_v.at[b]], gb, gsem)
                @pl.when(i >= 2)
                def _():
                    for _g in range(4):
                        drain(tok_hbm.at[idx_v.at[b]],
                              ob.at[pl.ds(0, 8), pl.ds(0, _NB)], osem)

                pv0 = pos_v[l, pl.ds(0, 16)]
                pv1 = pos_v[l, pl.ds(16, 16)]

                @pl.loop(0, _NB, step=8)
                def _rows(r0):
                    for u in range(8):
                        r = r0 + u
                        col = jnp.full((16,), r, jnp.int32)
                        v0 = gb[r, pl.ds(0, 16)] + pv0
                        v1 = gb[r, pl.ds(16, 16)] + pv1
                        plsc.store_scatter(ob, [row_lo, col], v0)
                        plsc.store_scatter(ob, [row_hi, col], v1)

                for g in range(4):
                    pltpu.async_copy(ob.at[pl.ds(8 * g, 8), pl.ds(0, _NB)],
                                     out_hbm.at[l, g, bg], osem)

                @pl.when(i + 2 < _IPW)
                def _():
                    fire_idx(i + 2, b)
                    drain(x_hbm.at[0, 0, 0], idx_v.at[b], isem)
                    fire_gather(b)

        for b in (0, 1):
            for _g in range(4):
                drain(tok_hbm.at[idx_v.at[b]],
                      out_v.at[b].at[pl.ds(0, 8), pl.ds(0, _NB)], osem)

    return k(xv, token_table, pos_table)


def kernel(x, token_table, pos_table):
    b, l = x.shape
    xi = x.astype(jnp.int32)
    xv = jnp.transpose(xi.reshape(_B // _NB, _NB, _LG, 8), (2, 0, 3, 1))
    out5 = _sc_embed(xv, token_table, pos_table)
    return jnp.transpose(out5, (2, 4, 0, 1, 3)).reshape(b, l, _D)


# speedup vs baseline: 1.4714x; 1.0006x over previous
import dataclasses
import functools

import jax
import jax.numpy as jnp
from jax import lax
from jax.experimental import pallas as pl
from jax.experimental.pallas import tpu as pltpu
from jax.experimental.pallas import tpu_sc as plsc

_L = 200
_D = 32
_B = 4096
_NW = 32
_NB = 128
_NITEMS = _L * (_B // _NB)
_IPW = _NITEMS // _NW
_LG = _L // 8


def _compiler_params():
    cp = pltpu.CompilerParams(use_tc_tiling_on_sc=False)
    if "needs_layout_passes" in pltpu.CompilerParams.__dataclass_fields__:
        cp = dataclasses.replace(cp, needs_layout_passes=False)
    return cp


def _sc_embed(xv, token_table, pos_table):
    mesh = plsc.VectorSubcoreMesh(core_axis_name="c", subcore_axis_name="s")

    @functools.partial(
        pl.kernel,
        out_type=jax.ShapeDtypeStruct((_L, _D // 8, _B // _NB, 8, _NB),
                                      jnp.float32),
        mesh=mesh,
        compiler_params=_compiler_params(),
        scratch_types=[
            pltpu.VMEM((_L, _D), jnp.float32),
            pltpu.VMEM((2, _NB), jnp.int32),
            pltpu.VMEM((2, _NB, _D), jnp.float32),
            pltpu.VMEM((2, _D, _NB + 1), jnp.float32),
            pltpu.SemaphoreType.DMA,
            pltpu.SemaphoreType.DMA,
            pltpu.SemaphoreType.DMA,
        ],
    )
    def k(x_hbm, tok_hbm, pos_hbm, out_hbm, pos_v, idx_v, g_v, out_v,
          isem, gsem, osem):
        wid = lax.axis_index("s") * 2 + lax.axis_index("c")
        q0 = wid * _IPW
        pltpu.sync_copy(pos_hbm, pos_v)

        row_lo = lax.iota(jnp.int32, 16)
        row_hi = row_lo + 16

        def item_coords(i):
            q = q0 + i
            l = q // (_B // _NB)
            bg = q % (_B // _NB)
            return l, bg

        def fire_idx(i, b):
            l, bg = item_coords(i)
            pltpu.async_copy(x_hbm.at[l // 8, bg, l % 8], idx_v.at[b], isem)

        def fire_gather(b):
            pltpu.async_copy(tok_hbm.at[idx_v.at[b]], g_v.at[b], gsem)

        def drain(dummy_src, dst, sem):
            pltpu.make_async_copy(dummy_src, dst, sem).wait()

        for b in (0, 1):
            fire_idx(b, b)
        for b in (0, 1):
            drain(x_hbm.at[0, 0, 0], idx_v.at[b], isem)
            fire_gather(b)

        @pl.loop(0, _IPW, step=2)
        def _slot(i0):
            for b in (0, 1):
                i = i0 + b
                l, bg = item_coords(i)
                gb = g_v.at[b]
                ob = out_v.at[b]

                drain(tok_hbm.at[idx_v.at[b]], gb, gsem)
                @pl.when(i >= 2)
                def _():
                    for _g in range(4):
                        drain(tok_hbm.at[idx_v.at[b]],
                              ob.at[pl.ds(0, 8), pl.ds(0, _NB)], osem)

                pv0 = pos_v[l, pl.ds(0, 16)]
                pv1 = pos_v[l, pl.ds(16, 16)]

                @pl.loop(0, _NB, step=8)
                def _rows(r0):
                    for u in range(8):
                        r = r0 + u
                        col = jnp.full((16,), r, jnp.int32)
                        v0 = gb[r, pl.ds(0, 16)] + pv0
                        v1 = gb[r, pl.ds(16, 16)] + pv1
                        plsc.store_scatter(ob, [row_lo, col], v0)
                        plsc.store_scatter(ob, [row_hi, col], v1)

                for g in range(4):
                    pltpu.async_copy(ob.at[pl.ds(8 * g, 8), pl.ds(0, _NB)],
                                     out_hbm.at[l, g, bg], osem)

                @pl.when(i + 2 < _IPW)
                def _():
                    fire_idx(i + 2, b)
                    drain(x_hbm.at[0, 0, 0], idx_v.at[b], isem)
                    fire_gather(b)

        for b in (0, 1):
            for _g in range(4):
                drain(tok_hbm.at[idx_v.at[b]],
                      out_v.at[b].at[pl.ds(0, 8), pl.ds(0, _NB)], osem)

    return k(xv, token_table, pos_table)


def kernel(x, token_table, pos_table):
    b, l = x.shape
    xi = x.astype(jnp.int32)
    xv = jnp.transpose(xi.reshape(_B // _NB, _NB, _LG, 8), (2, 0, 3, 1))
    nrow = token_table.shape[0] * _D // 128
    tt128 = jax.lax.optimization_barrier(jnp.reshape(token_table, (nrow, 128)))
    ttlin = jnp.reshape(tt128, token_table.shape)
    out5 = _sc_embed(xv, ttlin, pos_table)
    return jnp.transpose(out5, (2, 4, 0, 1, 3)).reshape(b, l, _D)


# speedup vs baseline: 2.0720x; 1.4082x over previous
import dataclasses
import functools

import jax
import jax.numpy as jnp
from jax import lax
from jax.experimental import pallas as pl
from jax.experimental.pallas import tpu as pltpu
from jax.experimental.pallas import tpu_sc as plsc

_L = 200
_D = 32
_B = 4096
_NW = 32
_NB = 128
_NITEMS = _L * (_B // _NB)
_IPW = _NITEMS // _NW
_LG = _L // 8


def _compiler_params():
    cp = pltpu.CompilerParams(use_tc_tiling_on_sc=False)
    if "needs_layout_passes" in pltpu.CompilerParams.__dataclass_fields__:
        cp = dataclasses.replace(cp, needs_layout_passes=False)
    return cp


_V = 1000000
_TCB = 1024
_TNB = _V // _TCB
_TTAIL = _V - _TNB * _TCB


def _sc_table_pack(ttT, tail):
    mesh = plsc.VectorSubcoreMesh(core_axis_name="c", subcore_axis_name="s")
    cp = pltpu.CompilerParams()
    if "needs_layout_passes" in pltpu.CompilerParams.__dataclass_fields__:
        cp = dataclasses.replace(cp, needs_layout_passes=False)

    @functools.partial(
        pl.kernel,
        out_type=jax.ShapeDtypeStruct((_V * _D // 128, 128), jnp.float32),
        mesh=mesh,
        compiler_params=cp,
        scratch_types=[
            pltpu.VMEM((2, _D, _TCB), jnp.float32),
            pltpu.VMEM((_TCB // 4, 132), jnp.float32),
            pltpu.SemaphoreType.DMA,
            pltpu.SemaphoreType.DMA,
        ],
    )
    def k(tt_hbm, tail_hbm, out_hbm, tin, tskew, isem, osem):
        wid = lax.axis_index("s") * 2 + lax.axis_index("c")
        iota = lax.iota(jnp.int32, 16)
        c33 = (iota & 3) * 33
        r4 = iota >> 2

        def fire_in(q, b):
            pltpu.async_copy(tt_hbm.at[:, pl.ds(q * _TCB, _TCB)],
                             tin.at[b], isem)

        def body(b, q, ntok):
            pltpu.make_async_copy(tt_hbm.at[:, pl.ds(0, _TCB)],
                                  tin.at[b], isem).wait()
            tb = tin.at[b]

            @pl.loop(0, ntok, step=16)
            def _grp(t0):
                rowv = r4 + (t0 // 4)
                for f in range(_D):
                    colv = c33 + f
                    plsc.store_scatter(tskew, [rowv, colv],
                                       tb[f, pl.ds(t0, 16)])

            pltpu.async_copy(tskew.at[pl.ds(0, ntok // 4), pl.ds(0, 128)],
                             out_hbm.at[pl.ds(q * (_TCB // 4), ntok // 4)],
                             osem)

        nmine = (_TNB - wid + _NW - 1) // _NW
        fire_in(wid, 0)

        @pl.loop(0, 31)
        def _blk(i):
            @pl.when(i < nmine)
            def _():
                q = wid + i * _NW
                b = i % 2

                @pl.when(i + 1 < nmine)
                def _():
                    fire_in(q + _NW, (i + 1) % 2)

                @pl.when(i >= 1)
                def _():
                    pltpu.make_async_copy(
                        tskew.at[pl.ds(0, _TCB // 4), pl.ds(0, 128)],
                        out_hbm.at[pl.ds(0, _TCB // 4)], osem).wait()

                body(b, q, _TCB)

        pltpu.make_async_copy(tskew.at[pl.ds(0, _TCB // 4), pl.ds(0, 128)],
                              out_hbm.at[pl.ds(0, _TCB // 4)], osem).wait()

        @pl.when(wid == 0)
        def _():
            pltpu.sync_copy(tail_hbm,
                            out_hbm.at[pl.ds(_TNB * (_TCB // 4), _TTAIL // 4)])

    return k(ttT, tail)


def _sc_embed(xv, token_table, pos_table):
    mesh = plsc.VectorSubcoreMesh(core_axis_name="c", subcore_axis_name="s")

    @functools.partial(
        pl.kernel,
        out_type=jax.ShapeDtypeStruct((_L, _D // 8, _B // _NB, 8, _NB),
                                      jnp.float32),
        mesh=mesh,
        compiler_params=_compiler_params(),
        scratch_types=[
            pltpu.VMEM((_L, _D), jnp.float32),
            pltpu.VMEM((2, _NB), jnp.int32),
            pltpu.VMEM((2, _NB, _D), jnp.float32),
            pltpu.VMEM((2, _D, _NB + 1), jnp.float32),
            pltpu.SemaphoreType.DMA,
            pltpu.SemaphoreType.DMA,
            pltpu.SemaphoreType.DMA,
        ],
    )
    def k(x_hbm, tok_hbm, pos_hbm, out_hbm, pos_v, idx_v, g_v, out_v,
          isem, gsem, osem):
        wid = lax.axis_index("s") * 2 + lax.axis_index("c")
        q0 = wid * _IPW
        pltpu.sync_copy(pos_hbm, pos_v)

        row_lo = lax.iota(jnp.int32, 16)
        row_hi = row_lo + 16

        def item_coords(i):
            q = q0 + i
            l = q // (_B // _NB)
            bg = q % (_B // _NB)
            return l, bg

        def fire_idx(i, b):
            l, bg = item_coords(i)
            pltpu.async_copy(x_hbm.at[l // 8, bg, l % 8], idx_v.at[b], isem)

        def fire_gather(b):
            pltpu.async_copy(tok_hbm.at[idx_v.at[b]], g_v.at[b], gsem)

        def drain(dummy_src, dst, sem):
            pltpu.make_async_copy(dummy_src, dst, sem).wait()

        for b in (0, 1):
            fire_idx(b, b)
        for b in (0, 1):
            drain(x_hbm.at[0, 0, 0], idx_v.at[b], isem)
            fire_gather(b)

        @pl.loop(0, _IPW, step=2)
        def _slot(i0):
            for b in (0, 1):
                i = i0 + b
                l, bg = item_coords(i)
                gb = g_v.at[b]
                ob = out_v.at[b]

                drain(tok_hbm.at[idx_v.at[b]], gb, gsem)
                @pl.when(i >= 2)
                def _():
                    for _g in range(4):
                        drain(tok_hbm.at[idx_v.at[b]],
                              ob.at[pl.ds(0, 8), pl.ds(0, _NB)], osem)

                pv0 = pos_v[l, pl.ds(0, 16)]
                pv1 = pos_v[l, pl.ds(16, 16)]

                @pl.loop(0, _NB, step=8)
                def _rows(r0):
                    for u in range(8):
                        r = r0 + u
                        col = jnp.full((16,), r, jnp.int32)
                        v0 = gb[r, pl.ds(0, 16)] + pv0
                        v1 = gb[r, pl.ds(16, 16)] + pv1
                        plsc.store_scatter(ob, [row_lo, col], v0)
                        plsc.store_scatter(ob, [row_hi, col], v1)

                for g in range(4):
                    pltpu.async_copy(ob.at[pl.ds(8 * g, 8), pl.ds(0, _NB)],
                                     out_hbm.at[l, g, bg], osem)

                @pl.when(i + 2 < _IPW)
                def _():
                    fire_idx(i + 2, b)
                    drain(x_hbm.at[0, 0, 0], idx_v.at[b], isem)
                    fire_gather(b)

        for b in (0, 1):
            for _g in range(4):
                drain(tok_hbm.at[idx_v.at[b]],
                      out_v.at[b].at[pl.ds(0, 8), pl.ds(0, _NB)], osem)

    return k(xv, token_table, pos_table)


def kernel(x, token_table, pos_table):
    b, l = x.shape
    xi = x.astype(jnp.int32)
    xv = jnp.transpose(xi.reshape(_B // _NB, _NB, _LG, 8), (2, 0, 3, 1))
    tail = jnp.reshape(token_table[_TNB * _TCB:, :], (_TTAIL // 4, 128))
    tt128 = _sc_table_pack(jnp.transpose(token_table), tail)
    ttlin = jnp.reshape(tt128, token_table.shape)
    out5 = _sc_embed(xv, ttlin, pos_table)
    return jnp.transpose(out5, (2, 4, 0, 1, 3)).reshape(b, l, _D)
